# Initial kernel scaffold; baseline (speedup 1.0000x reference)
import jax
import jax.numpy as jnp
from jax.experimental import pallas as pl


def kernel(x, edge_index, edge_attr, W1, b1, W2, b2, W3, b3, W4, b4, root, conv_bias, F1w, F1b, bn1g, bn1b, bn1m, bn1v, F2w, F2b, bn2g, bn2b, bn2m, bn2v, F3w, F3b):
    raise NotImplementedError("write your pallas kernel here")


# speedup vs baseline: 6.0283x; 6.0283x over previous
import functools

import jax
import jax.numpy as jnp
from jax import lax
from jax.experimental import pallas as pl
from jax.experimental.pallas import tpu as pltpu
from jax.experimental.pallas import tpu_sc as plsc

_NC = 2
_NS = 16
_PACK = 10
_BR = 4000
_BE = 6400
_HI = lax.Precision.HIGHEST


def _sc_edge_weights(E, N):
    NW = _NC * _NS
    ept_h = E // _NS
    ept = E // NW
    ch = 4000
    cp = 2000
    npad = -(-N // (16 * _NS)) * (16 * _NS)
    sl = npad // _NS

    mesh = plsc.VectorSubcoreMesh(core_axis_name="c", subcore_axis_name="s",
                                  num_cores=_NC, num_subcores=_NS)

    @functools.partial(
        pl.kernel,
        out_type=jax.ShapeDtypeStruct((3 * E,), jnp.float32),
        mesh=mesh,
        compiler_params=pltpu.CompilerParams(needs_layout_passes=False),
        scratch_types=[
            pltpu.VMEM((3 * N,), jnp.float32),
            pltpu.VMEM((npad,), jnp.float32),
            pltpu.VMEM((ch,), jnp.int32),
            pltpu.VMEM((ch,), jnp.float32),
            pltpu.VMEM((cp,), jnp.int32),
            pltpu.VMEM((cp,), jnp.int32),
            pltpu.VMEM((3 * cp,), jnp.float32),
            pltpu.VMEM((sl,), jnp.float32),
            pltpu.VMEM_SHARED((npad,), jnp.float32),
        ],
    )
    def sc_k(src_hbm, dst_hbm, xf_hbm, a_hbm,
             xloc, invloc, dsth, ones, srcb, dstb, outb, tmp, cnt_sh):
        sid = lax.axis_index("s")
        cid = lax.axis_index("c")
        wid = sid * _NC + cid

        for i in range(sl // 16):
            tmp[pl.ds(16 * i, 16)] = jnp.zeros((16,), jnp.float32)
        pltpu.sync_copy(tmp, cnt_sh.at[pl.ds(sid * sl, sl)])

        def fill_ones(i, c):
            ones[pl.ds(16 * i, 16)] = jnp.full((16,), 1.0, jnp.float32)
            return c
        lax.fori_loop(0, ch // 16, fill_ones, 0)
        plsc.subcore_barrier()

        hbase = sid * ept_h
        for c in range(ept_h // ch):
            pltpu.sync_copy(dst_hbm.at[pl.ds(hbase + c * ch, ch)], dsth)
            pltpu.sync_copy(ones, cnt_sh.at[dsth], add=True)
        plsc.subcore_barrier()

        pltpu.sync_copy(cnt_sh.at[pl.ds(sid * sl, sl)], tmp)
        for i in range(sl // 16):
            v = tmp[pl.ds(16 * i, 16)]
            tmp[pl.ds(16 * i, 16)] = 1.0 / jnp.maximum(v, 1.0)
        pltpu.sync_copy(tmp, cnt_sh.at[pl.ds(sid * sl, sl)])
        plsc.subcore_barrier()

        pltpu.sync_copy(cnt_sh, invloc)
        pltpu.sync_copy(xf_hbm, xloc)

        lane3 = lax.iota(jnp.int32, 16) * 3
        base = wid * ept
        for c in range(ept // cp):
            off = base + c * cp
            pltpu.sync_copy(src_hbm.at[pl.ds(off, cp)], srcb)
            pltpu.sync_copy(dst_hbm.at[pl.ds(off, cp)], dstb)

            def grp(j, carry):
                s3 = srcb[pl.ds(j * 16, 16)] * 3
                d_v = dstb[pl.ds(j * 16, 16)]
                inv_v = plsc.load_gather(invloc, [d_v])
                p0 = j * 48 + lane3
                plsc.store_scatter(outb, [p0],
                                   plsc.load_gather(xloc, [s3]) * inv_v)
                plsc.store_scatter(outb, [p0 + 1],
                                   plsc.load_gather(xloc, [s3 + 1]) * inv_v)
                plsc.store_scatter(outb, [p0 + 2],
                                   plsc.load_gather(xloc, [s3 + 2]) * inv_v)
                return carry
            lax.fori_loop(0, cp // 16, grp, 0)
            pltpu.sync_copy(outb, a_hbm.at[pl.ds(3 * off, 3 * cp)])

    return sc_k


def _reduce_head_body(ea, ap, w1, b1, w2, b2, w3, b3, w4a, w4b, w4c, b4r,
                      xr, rootr, cbr, f1w, f1b, g1, be1, m1, v1,
                      f2w, f2b, g2, be2, m2, v2, f3w, f3b,
                      y_out, x2_out, gacc):
    i = pl.program_id(0)

    @pl.when(i == 0)
    def _init():
        gacc[...] = jnp.zeros_like(gacc)

    h = jnp.maximum(jnp.dot(ea[...], w1[...],
                            preferred_element_type=jnp.float32) + b1[...], 0.0)
    h = jnp.maximum(jnp.dot(h, w2[...],
                            preferred_element_type=jnp.float32) + b2[...], 0.0)
    h = jnp.maximum(jnp.dot(h, w3[...],
                            preferred_element_type=jnp.float32) + b3[...], 0.0)
    gacc[...] += lax.dot_general(ap[...], h, (((0,), (0,)), ((), ())),
                                 preferred_element_type=jnp.float32,
                                 precision=_HI)

    @pl.when(i == pl.num_programs(0) - 1)
    def _fin():
        n_nodes = x2_out.shape[0]
        gp = gacc[...]
        g3 = jnp.zeros((3, 20), jnp.float32)
        s3 = jnp.zeros((3, 1), jnp.float32)
        for p in range(_PACK):
            g3 = g3 + gp[3 * p:3 * p + 3, 20 * p:20 * p + 20]
            s3 = s3 + gp[3 * p:3 * p + 3, 200:201]
        msg = (jnp.dot(g3[0:1], w4a[...], precision=_HI,
                       preferred_element_type=jnp.float32)
               + jnp.dot(g3[1:2], w4b[...], precision=_HI,
                         preferred_element_type=jnp.float32)
               + jnp.dot(g3[2:3], w4c[...], precision=_HI,
                         preferred_element_type=jnp.float32)
               + s3[0:1, 0:1] * b4r[0:1, :]
               + s3[1:2, 0:1] * b4r[1:2, :]
               + s3[2:3, 0:1] * b4r[2:3, :])
        xmean = jnp.sum(xr[...], axis=0, keepdims=True) * (1.0 / n_nodes)
        xm = (msg * (1.0 / n_nodes)
              + jnp.dot(xmean, rootr[...], precision=_HI,
                        preferred_element_type=jnp.float32)
              + cbr[...])
        t = jnp.dot(xm, f1w[...], precision=_HI,
                    preferred_element_type=jnp.float32) + f1b[...]
        t = (t - m1[...]) / jnp.sqrt(v1[...] + 1e-5) * g1[...] + be1[...]
        t = jnp.maximum(t, 0.0)
        t = jnp.dot(t, f2w[...], precision=_HI,
                    preferred_element_type=jnp.float32) + f2b[...]
        t = (t - m2[...]) / jnp.sqrt(v2[...] + 1e-5) * g2[...] + be2[...]
        t = jnp.maximum(t, 0.0)
        y = jnp.dot(t, f3w[...], precision=_HI,
                    preferred_element_type=jnp.float32) + f3b[...]
        y_out[...] = y
        x2_out[...] = jnp.broadcast_to(y, x2_out.shape)


def _nea_body(ea, yb, out):
    out[...] = jnp.concatenate(
        [ea[...][:, 0:6], jnp.broadcast_to(yb[...], (ea.shape[0], 50))],
        axis=1)


def _full(shape):
    return pl.BlockSpec(shape, lambda i: tuple(0 for _ in shape))


def kernel(x, edge_index, edge_attr, W1, b1, W2, b2, W3, b3, W4, b4,
           root, conv_bias, F1w, F1b, bn1g, bn1b, bn1m, bn1v,
           F2w, F2b, bn2g, bn2b, bn2m, bn2v, F3w, F3b):
    N, IC = x.shape
    E = edge_attr.shape[0]
    L = root.shape[1]

    a_flat = _sc_edge_weights(E, N)(edge_index[0], edge_index[1],
                                    x.reshape(-1))
    ap = a_flat.reshape(E // _PACK, 3 * _PACK)

    import jax.scipy.linalg as jsl
    w1bd = jsl.block_diag(*([W1] * _PACK))
    w2bd = jsl.block_diag(*([W2] * _PACK))
    w3bd = jnp.concatenate(
        [jsl.block_diag(*([W3] * _PACK)), jnp.zeros((20 * _PACK, 1))],
        axis=1)
    b1t = jnp.tile(b1, _PACK)[None, :]
    b2t = jnp.tile(b2, _PACK)[None, :]
    b3t = jnp.concatenate([jnp.tile(b3, _PACK), jnp.ones((1,))])[None, :]
    b4r = b4.reshape(IC, L)
    eap = edge_attr.reshape(E // _PACK, 12 * _PACK)

    rows = E // _PACK
    grid = rows // _BR
    y, x_2 = pl.pallas_call(
        _reduce_head_body,
        grid=(grid,),
        in_specs=[
            pl.BlockSpec((_BR, 12 * _PACK), lambda i: (i, 0)),
            pl.BlockSpec((_BR, 3 * _PACK), lambda i: (i, 0)),
            _full(w1bd.shape), _full(b1t.shape),
            _full(w2bd.shape), _full(b2t.shape),
            _full(w3bd.shape), _full(b3t.shape),
            _full((20, L)), _full((20, L)), _full((20, L)), _full((IC, L)),
            _full(x.shape), _full(root.shape), _full((1, L)),
            _full(F1w.shape), _full((1, L)), _full((1, L)), _full((1, L)),
            _full((1, L)), _full((1, L)),
            _full(F2w.shape), _full((1, L)), _full((1, L)), _full((1, L)),
            _full((1, L)), _full((1, L)),
            _full(F3w.shape), _full((1, L)),
        ],
        out_specs=[
            pl.BlockSpec((1, L), lambda i: (0, 0)),
            pl.BlockSpec((N, L), lambda i: (0, 0)),
        ],
        out_shape=[
            jax.ShapeDtypeStruct((1, L), jnp.float32),
            jax.ShapeDtypeStruct((N, L), jnp.float32),
        ],
        scratch_shapes=[pltpu.VMEM((3 * _PACK, 20 * _PACK + 1), jnp.float32)],
    )(eap, ap, w1bd, b1t, w2bd, b2t, w3bd, b3t,
      W4[:, 0:L], W4[:, L:2 * L], W4[:, 2 * L:3 * L], b4r,
      x, root, conv_bias[None, :],
      F1w, F1b[None, :], bn1g[None, :], bn1b[None, :], bn1m[None, :],
      bn1v[None, :],
      F2w, F2b[None, :], bn2g[None, :], bn2b[None, :], bn2m[None, :],
      bn2v[None, :],
      F3w, F3b[None, :])

    new_edge_attr = pl.pallas_call(
        _nea_body,
        grid=(E // _BE,),
        in_specs=[
            pl.BlockSpec((_BE, 12), lambda i: (i, 0)),
            pl.BlockSpec((1, L), lambda i: (0, 0)),
        ],
        out_specs=pl.BlockSpec((_BE, 6 + L), lambda i: (i, 0)),
        out_shape=jax.ShapeDtypeStruct((E, 6 + L), jnp.float32),
    )(edge_attr, y)

    return (x_2, new_edge_attr, y)


# speedup vs baseline: 23.9808x; 3.9780x over previous
import functools

import jax
import jax.numpy as jnp
from jax import lax
from jax.experimental import pallas as pl
from jax.experimental.pallas import tpu as pltpu
from jax.experimental.pallas import tpu_sc as plsc

_NC = 2
_NS = 16
_BER = 25600
_BEN = 25600
_HI = lax.Precision.HIGHEST


def _sc_edge_weights(E, N):
    NW = _NC * _NS
    ept_h = E // _NS
    ept = E // NW
    ch = 4000
    cp = 2000
    npad = -(-N // (16 * _NS)) * (16 * _NS)
    sl = npad // _NS

    mesh = plsc.VectorSubcoreMesh(core_axis_name="c", subcore_axis_name="s",
                                  num_cores=_NC, num_subcores=_NS)

    @functools.partial(
        pl.kernel,
        out_type=jax.ShapeDtypeStruct((3 * E,), jnp.float32),
        mesh=mesh,
        compiler_params=pltpu.CompilerParams(needs_layout_passes=False),
        scratch_types=[
            pltpu.VMEM((3 * N,), jnp.float32),
            pltpu.VMEM((npad,), jnp.float32),
            pltpu.VMEM((ch,), jnp.int32),
            pltpu.VMEM((ch,), jnp.float32),
            pltpu.VMEM((cp,), jnp.int32),
            pltpu.VMEM((cp,), jnp.int32),
            pltpu.VMEM((cp,), jnp.float32),
            pltpu.VMEM((cp,), jnp.float32),
            pltpu.VMEM((cp,), jnp.float32),
            pltpu.VMEM((sl,), jnp.float32),
            pltpu.VMEM_SHARED((npad,), jnp.float32),
        ],
    )
    def sc_k(src_hbm, dst_hbm, xt_hbm, a_hbm,
             xloc, invloc, dsth, ones, srcb, dstb, ob0, ob1, ob2, tmp,
             cnt_sh):
        sid = lax.axis_index("s")
        cid = lax.axis_index("c")
        wid = sid * _NC + cid

        for i in range(sl // 16):
            tmp[pl.ds(16 * i, 16)] = jnp.zeros((16,), jnp.float32)
        pltpu.sync_copy(tmp, cnt_sh.at[pl.ds(sid * sl, sl)])

        def fill_ones(i, c):
            ones[pl.ds(16 * i, 16)] = jnp.full((16,), 1.0, jnp.float32)
            return c
        lax.fori_loop(0, ch // 16, fill_ones, 0)
        plsc.subcore_barrier()

        hbase = sid * ept_h
        for c in range(ept_h // ch):
            pltpu.sync_copy(dst_hbm.at[pl.ds(hbase + c * ch, ch)], dsth)
            pltpu.sync_copy(ones, cnt_sh.at[dsth], add=True)
        plsc.subcore_barrier()

        pltpu.sync_copy(cnt_sh.at[pl.ds(sid * sl, sl)], tmp)
        for i in range(sl // 16):
            v = tmp[pl.ds(16 * i, 16)]
            tmp[pl.ds(16 * i, 16)] = 1.0 / jnp.maximum(v, 1.0)
        pltpu.sync_copy(tmp, cnt_sh.at[pl.ds(sid * sl, sl)])
        plsc.subcore_barrier()

        pltpu.sync_copy(cnt_sh, invloc)
        pltpu.sync_copy(xt_hbm, xloc)

        base = wid * ept
        for c in range(ept // cp):
            off = base + c * cp
            pltpu.sync_copy(src_hbm.at[pl.ds(off, cp)], srcb)
            pltpu.sync_copy(dst_hbm.at[pl.ds(off, cp)], dstb)

            def grp(j, carry):
                s_v = srcb[pl.ds(j * 16, 16)]
                d_v = dstb[pl.ds(j * 16, 16)]
                inv_v = plsc.load_gather(invloc, [d_v])
                ob0[pl.ds(j * 16, 16)] = (
                    plsc.load_gather(xloc, [s_v]) * inv_v)
                ob1[pl.ds(j * 16, 16)] = (
                    plsc.load_gather(xloc, [s_v + N]) * inv_v)
                ob2[pl.ds(j * 16, 16)] = (
                    plsc.load_gather(xloc, [s_v + 2 * N]) * inv_v)
                return carry
            lax.fori_loop(0, cp // 16, grp, 0)
            pltpu.sync_copy(ob0, a_hbm.at[pl.ds(off, cp)])
            pltpu.sync_copy(ob1, a_hbm.at[pl.ds(E + off, cp)])
            pltpu.sync_copy(ob2, a_hbm.at[pl.ds(2 * E + off, cp)])

    return sc_k


def _reduce_head_body(ea, a0, a1, a2, w1t, b1c, w2t, b2c, w3te, b3ce,
                      w4at, w4bt, w4ct, b4t, xt, roott, cbc,
                      f1wt, f1bc, g1c, be1c, m1c, v1c,
                      f2wt, f2bc, g2c, be2c, m2c, v2c, f3wt, f3bc,
                      y_out, x2_out, gacc):
    i = pl.program_id(0)

    @pl.when(i == 0)
    def _init():
        gacc[...] = jnp.zeros_like(gacc)

    h = jnp.maximum(jnp.dot(w1t[...], ea[...],
                            preferred_element_type=jnp.float32) + b1c[...],
                    0.0)
    h = jnp.maximum(jnp.dot(w2t[...], h,
                            preferred_element_type=jnp.float32) + b2c[...],
                    0.0)
    h = jnp.maximum(jnp.dot(w3te[...], h,
                            preferred_element_type=jnp.float32) + b3ce[...],
                    0.0)
    at = jnp.concatenate([a0[...].reshape(1, -1), a1[...].reshape(1, -1),
                          a2[...].reshape(1, -1)], axis=0)
    gacc[...] += lax.dot_general(h, at, (((1,), (1,)), ((), ())),
                                 preferred_element_type=jnp.float32,
                                 precision=_HI)

    @pl.when(i == pl.num_programs(0) - 1)
    def _fin():
        n_nodes = x2_out.shape[1]
        g3t = gacc[0:20, :]
        s3t = gacc[20:21, :]
        msg = (jnp.dot(w4at[...], g3t[:, 0:1], precision=_HI,
                       preferred_element_type=jnp.float32)
               + jnp.dot(w4bt[...], g3t[:, 1:2], precision=_HI,
                         preferred_element_type=jnp.float32)
               + jnp.dot(w4ct[...], g3t[:, 2:3], precision=_HI,
                         preferred_element_type=jnp.float32)
               + s3t[0:1, 0:1] * b4t[:, 0:1]
               + s3t[0:1, 1:2] * b4t[:, 1:2]
               + s3t[0:1, 2:3] * b4t[:, 2:3])
        xmean = jnp.sum(xt[...], axis=1, keepdims=True) * (1.0 / n_nodes)
        xm = (msg * (1.0 / n_nodes)
              + jnp.dot(roott[...], xmean, precision=_HI,
                        preferred_element_type=jnp.float32)
              + cbc[...])
        t = jnp.dot(f1wt[...], xm, precision=_HI,
                    preferred_element_type=jnp.float32) + f1bc[...]
        t = (t - m1c[...]) / jnp.sqrt(v1c[...] + 1e-5) * g1c[...] + be1c[...]
        t = jnp.maximum(t, 0.0)
        t = jnp.dot(f2wt[...], t, precision=_HI,
                    preferred_element_type=jnp.float32) + f2bc[...]
        t = (t - m2c[...]) / jnp.sqrt(v2c[...] + 1e-5) * g2c[...] + be2c[...]
        t = jnp.maximum(t, 0.0)
        y = jnp.dot(f3wt[...], t, precision=_HI,
                    preferred_element_type=jnp.float32) + f3bc[...]
        y_out[...] = y
        x2_out[...] = jnp.broadcast_to(y, x2_out.shape)


def _nea_body(ea, yc, out):
    out[...] = jnp.concatenate(
        [ea[0:6, :], jnp.broadcast_to(yc[...], (50, ea.shape[1]))], axis=0)


def _full(shape):
    return pl.BlockSpec(shape, lambda i: tuple(0 for _ in shape))


def kernel(x, edge_index, edge_attr, W1, b1, W2, b2, W3, b3, W4, b4,
           root, conv_bias, F1w, F1b, bn1g, bn1b, bn1m, bn1v,
           F2w, F2b, bn2g, bn2b, bn2m, bn2v, F3w, F3b):
    N, IC = x.shape
    E = edge_attr.shape[0]
    L = root.shape[1]

    xt = jnp.swapaxes(x, 0, 1)
    eat = jnp.swapaxes(edge_attr, 0, 1)

    a_flat = _sc_edge_weights(E, N)(edge_index[0], edge_index[1],
                                    xt.reshape(-1))
    a0, a1, a2 = a_flat[0:E], a_flat[E:2 * E], a_flat[2 * E:3 * E]

    w3te = jnp.concatenate([W3.T, jnp.zeros((1, 20))], axis=0)
    b3ce = jnp.concatenate([b3, jnp.ones((1,))])[:, None]

    grid = E // _BER
    y, x2t = pl.pallas_call(
        _reduce_head_body,
        grid=(grid,),
        in_specs=[
            pl.BlockSpec((12, _BER), lambda i: (0, i)),
            pl.BlockSpec((_BER,), lambda i: (i,)),
            pl.BlockSpec((_BER,), lambda i: (i,)),
            pl.BlockSpec((_BER,), lambda i: (i,)),
            _full((20, 12)), _full((20, 1)),
            _full((20, 20)), _full((20, 1)),
            _full((21, 20)), _full((21, 1)),
            _full((L, 20)), _full((L, 20)), _full((L, 20)), _full((L, IC)),
            _full((IC, N)), _full((L, IC)), _full((L, 1)),
            _full((L, L)), _full((L, 1)), _full((L, 1)), _full((L, 1)),
            _full((L, 1)), _full((L, 1)),
            _full((L, L)), _full((L, 1)), _full((L, 1)), _full((L, 1)),
            _full((L, 1)), _full((L, 1)),
            _full((L, L)), _full((L, 1)),
        ],
        out_specs=[
            pl.BlockSpec((L, 1), lambda i: (0, 0)),
            pl.BlockSpec((L, N), lambda i: (0, 0)),
        ],
        out_shape=[
            jax.ShapeDtypeStruct((L, 1), jnp.float32),
            jax.ShapeDtypeStruct((L, N), jnp.float32),
        ],
        scratch_shapes=[pltpu.VMEM((21, 3), jnp.float32)],
    )(eat, a0, a1, a2, W1.T, b1[:, None], W2.T, b2[:, None], w3te, b3ce,
      W4[:, 0:L].T, W4[:, L:2 * L].T, W4[:, 2 * L:3 * L].T,
      b4.reshape(IC, L).T,
      xt, root.T, conv_bias[:, None],
      F1w.T, F1b[:, None], bn1g[:, None], bn1b[:, None], bn1m[:, None],
      bn1v[:, None],
      F2w.T, F2b[:, None], bn2g[:, None], bn2b[:, None], bn2m[:, None],
      bn2v[:, None],
      F3w.T, F3b[:, None])

    neat = pl.pallas_call(
        _nea_body,
        grid=(E // _BEN,),
        in_specs=[
            pl.BlockSpec((12, _BEN), lambda i: (0, i)),
            pl.BlockSpec((L, 1), lambda i: (0, 0)),
        ],
        out_specs=pl.BlockSpec((6 + L, _BEN), lambda i: (0, i)),
        out_shape=jax.ShapeDtypeStruct((6 + L, E), jnp.float32),
    )(eat, y)

    return (jnp.swapaxes(x2t, 0, 1), jnp.swapaxes(neat, 0, 1),
            jnp.swapaxes(y, 0, 1))
